# Initial kernel scaffold; baseline (speedup 1.0000x reference)
import jax
import jax.numpy as jnp
from jax.experimental import pallas as pl


def kernel(x):
    raise NotImplementedError("write your pallas kernel here")


# speedup vs baseline: 8.8290x; 8.8290x over previous
import functools

import jax
import jax.numpy as jnp
from jax import lax
from jax.experimental import pallas as pl
from jax.experimental.pallas import tpu as pltpu, tpu_sc as plsc

N = 50000
NS = 2048
NTILES = 16
NPAD = 50176
PPT = NPAD // NTILES
CHUNKS = PPT // 16
IMAX = 2147483647

_mesh = plsc.VectorSubcoreMesh(
    core_axis_name="c", subcore_axis_name="s", num_cores=1
)


@functools.partial(
    pl.kernel,
    mesh=_mesh,
    out_type=jax.ShapeDtypeStruct((NS, 1, 16), jnp.float32),
    compiler_params=pltpu.CompilerParams(
        needs_layout_passes=False, use_tc_tiling_on_sc=False),
    scratch_types=[
        pltpu.VMEM((3, PPT), jnp.float32),
        pltpu.VMEM((1, PPT), jnp.float32),
        pltpu.VMEM((1, 1, 16), jnp.float32),
        pltpu.VMEM((16, 1, 32), jnp.float32),
        pltpu.VMEM((1, 32), jnp.float32),
        pltpu.VMEM((NS, 1, 16), jnp.float32),
        pltpu.VMEM_SHARED((NPAD, 1, 16), jnp.float32),
        pltpu.VMEM_SHARED((32, 1, 32), jnp.float32),
    ],
)
def _fps_sc(xt_hbm, dinit_hbm, xr_hbm, out_hbm,
            xloc, dists, prow, redbuf, recstage, outbuf, xsh, recs):
    sid = lax.axis_index("s")

    pltpu.sync_copy(xt_hbm.at[sid], xloc)
    pltpu.sync_copy(dinit_hbm.at[sid], dists)

    @pl.when(sid == 0)
    def _():
        pltpu.sync_copy(xr_hbm, xsh)

    plsc.subcore_barrier()

    pltpu.sync_copy(xsh.at[pl.ds(0, 1)], prow)

    @pl.when(sid == 0)
    def _():
        pltpu.sync_copy(xsh.at[pl.ds(0, 1)], outbuf.at[pl.ds(0, 1)])

    base = sid * PPT
    lanes = lax.iota(jnp.int32, 16)
    zeros16 = jnp.zeros((16,), jnp.int32)
    ones16 = jnp.full((16,), 1, jnp.int32)
    twos16 = jnp.full((16,), 2, jnp.int32)

    def iter_body(i, _):
        pvec = prow[0, 0, :]
        p0 = jnp.full((16,), pvec[0], jnp.float32)
        p1 = jnp.full((16,), pvec[1], jnp.float32)
        p2 = jnp.full((16,), pvec[2], jnp.float32)

        def chunk_body(c, carry):
            runmax, runidx = carry
            off = c * 16
            t0 = xloc[0, pl.ds(off, 16)] - p0
            t1 = xloc[1, pl.ds(off, 16)] - p1
            t2 = xloc[2, pl.ds(off, 16)] - p2
            d = (t0 * t0 + t1 * t1) + t2 * t2
            dn = jnp.minimum(dists[0, pl.ds(off, 16)], d)
            dists[0, pl.ds(off, 16)] = dn
            upd = dn > runmax
            runmax = jnp.maximum(runmax, dn)
            runidx = jnp.where(upd, (base + off) + lanes, runidx)
            return runmax, runidx

        runmax, runidx = lax.fori_loop(
            0, CHUNKS, chunk_body,
            (jnp.full((16,), -jnp.inf, jnp.float32), zeros16),
        )

        m = jnp.max(runmax)
        mvec = jnp.full((16,), m, jnp.float32)
        li = jnp.min(jnp.where(runmax == mvec, runidx,
                               jnp.full((16,), IMAX, jnp.int32)))
        recstage[0, pl.ds(0, 16)] = mvec
        recstage[0, pl.ds(16, 16)] = plsc.bitcast(
            jnp.full((16,), li, jnp.int32), jnp.float32)

        pbase = jnp.bitwise_and(i, 1) * NTILES
        pltpu.sync_copy(recstage, recs.at[pbase + sid])
        plsc.subcore_barrier()
        pltpu.sync_copy(recs.at[pl.ds(pbase, NTILES)], redbuf)

        gm = redbuf[0, 0, pl.ds(0, 16)]
        for t in range(1, NTILES):
            gm = jnp.maximum(gm, redbuf[t, 0, pl.ds(0, 16)])
        best = jnp.full((16,), IMAX, jnp.int32)
        for t in range(NTILES):
            vt = redbuf[t, 0, pl.ds(0, 16)]
            it = plsc.bitcast(redbuf[t, 0, pl.ds(16, 16)], jnp.int32)
            best = jnp.where(vt == gm, jnp.minimum(best, it), best)
        bi = jnp.min(best)

        pltpu.sync_copy(xsh.at[pl.ds(bi, 1)], prow)

        @pl.when(sid == 0)
        def _():
            pltpu.sync_copy(xsh.at[pl.ds(bi, 1)], outbuf.at[pl.ds(i, 1)])

        return 0

    lax.fori_loop(1, NS, iter_body, 0)

    @pl.when(sid == 0)
    def _():
        pltpu.sync_copy(outbuf, out_hbm)


def kernel(x):
    xpad = jnp.pad(x, ((0, NPAD - N), (0, 0)))
    xt = xpad.T.reshape(3, NTILES, PPT).transpose(1, 0, 2)
    xr = jnp.pad(xpad, ((0, 0), (0, 13))).reshape(NPAD, 1, 16)
    dinit = jnp.concatenate([
        jnp.full((N,), jnp.inf, jnp.float32),
        jnp.full((NPAD - N,), -jnp.inf, jnp.float32),
    ]).reshape(NTILES, 1, PPT)
    out = _fps_sc(xt, dinit, xr)
    return out.reshape(NS, 16)[:, :3]
